# baseline (speedup 1.0000x reference)
import functools

import jax
import jax.numpy as jnp
from jax import lax
from jax.experimental import pallas as pl
from jax.experimental.pallas import tpu as pltpu
from jax.experimental.pallas import tpu_sc as plsc

N_NODES = 100000
N_EDGES = 6400000

NC = 2
NS = 16
L = 16
NW = NC * NS
E_PER_W = N_EDGES // NW
CHUNK = 4000
WIN = 4096
N_CHUNKS = E_PER_W // CHUNK
SL = 6240
TAIL = N_NODES - SL * NS

_mesh = plsc.VectorSubcoreMesh(
    core_axis_name="c", subcore_axis_name="s", num_cores=NC, num_subcores=NS
)


@functools.partial(
    pl.kernel,
    mesh=_mesh,
    out_type=jax.ShapeDtypeStruct((N_EDGES,), jnp.float32),
    compiler_params=pltpu.CompilerParams(needs_layout_passes=False),
    scratch_types=[
        pltpu.VMEM((N_NODES,), jnp.float32),
        pltpu.VMEM((2, WIN), jnp.int32),
        pltpu.VMEM((2, WIN), jnp.int32),
        pltpu.VMEM((CHUNK,), jnp.float32),
        pltpu.VMEM((CHUNK,), jnp.float32),
        pltpu.VMEM_SHARED((N_NODES,), jnp.float32),
        pltpu.SemaphoreType.DMA,
        pltpu.SemaphoreType.DMA,
        pltpu.SemaphoreType.DMA,
        pltpu.SemaphoreType.DMA,
    ],
)
def _edge_attn_sc(logits_hbm, ei_hbm, out_hbm,
                  tbl_v, idx_v0, idx_v1, out_v0, out_v1, attn_sh,
                  si0, si1, so0, so1):
    sid = lax.axis_index("s")
    wid = sid * NC + lax.axis_index("c")

    base_w = wid * E_PER_W
    idxs = (idx_v0, idx_v1)
    outs = (out_v0, out_v1)
    sin = (si0, si1)
    sout = (so0, so1)

    def start_in(b, c):
        base = base_w + c * CHUNK
        base_al = pl.multiple_of((base // 128) * 128, 128)
        pltpu.async_copy(ei_hbm.at[:, pl.ds(base_al, WIN)], idxs[b], sin[b])

    def wait_in(b):
        pltpu.make_async_copy(ei_hbm.at[:, pl.ds(0, WIN)], idxs[b], sin[b]).wait()

    def wait_out(b):
        pltpu.make_async_copy(outs[b], out_hbm.at[pl.ds(0, CHUNK)], sout[b]).wait()

    start_in(0, 0)

    s0 = pl.multiple_of(sid * SL, 8)
    pltpu.sync_copy(logits_hbm.at[pl.ds(s0, SL)], tbl_v.at[pl.ds(0, SL)])

    @plsc.parallel_loop(0, SL, step=L, unroll=4)
    def _sig(i):
        x = tbl_v[pl.ds(i, L)]
        tbl_v[pl.ds(i, L)] = 1.0 / (1.0 + jnp.exp(-x))

    pltpu.sync_copy(tbl_v.at[pl.ds(0, SL)], attn_sh.at[pl.ds(s0, SL)])

    @pl.when(sid == NS - 1)
    def _tail():
        pltpu.sync_copy(logits_hbm.at[pl.ds(SL * NS, TAIL)],
                        tbl_v.at[pl.ds(SL, TAIL)])

        @plsc.parallel_loop(SL, SL + TAIL, step=L, unroll=2)
        def _sig_t(i):
            x = tbl_v[pl.ds(i, L)]
            tbl_v[pl.ds(i, L)] = 1.0 / (1.0 + jnp.exp(-x))

        pltpu.sync_copy(tbl_v.at[pl.ds(SL, TAIL)],
                        attn_sh.at[pl.ds(SL * NS, TAIL)])

    plsc.subcore_barrier()
    pltpu.sync_copy(attn_sh, tbl_v)

    def pair_body(p, carry):
        for b in range(2):
            c = 2 * p + b
            cn = jnp.minimum(c + 1, N_CHUNKS - 1)
            start_in(1 - b, cn)
            wait_in(b)
            @pl.when(c >= 2)
            def _():
                wait_out(b)

            iv = idxs[b]
            ov = outs[b]
            base = base_w + c * CHUNK
            off = base - (base // 128) * 128

            @plsc.parallel_loop(0, CHUNK, step=L, unroll=16)
            def _gather(j):
                s = plsc.load_gather(tbl_v, [iv[0, pl.ds(off + j, L)]])
                d = plsc.load_gather(tbl_v, [iv[1, pl.ds(off + j, L)]])
                ov[pl.ds(j, L)] = s * d

            pltpu.async_copy(outs[b], out_hbm.at[pl.ds(base, CHUNK)], sout[b])
        return carry

    lax.fori_loop(0, N_CHUNKS // 2, pair_body, 0)
    wait_out(0)
    wait_out(1)
    wait_in(0)


def kernel(ver_logits, edge_index):
    return _edge_attn_sc(ver_logits, edge_index)


# speedup vs baseline: 1.0377x; 1.0377x over previous
import functools

import jax
import jax.numpy as jnp
from jax import lax
from jax.experimental import pallas as pl
from jax.experimental.pallas import tpu as pltpu
from jax.experimental.pallas import tpu_sc as plsc

N_NODES = 100000
N_EDGES = 6400000

NC = 2
NS = 16
L = 16
NW = NC * NS
E_PER_W = N_EDGES // NW
CHUNK = 4000
WIN = 4096
N_CHUNKS = E_PER_W // CHUNK
SL = 6240
TAIL = N_NODES - SL * NS

_mesh = plsc.VectorSubcoreMesh(
    core_axis_name="c", subcore_axis_name="s", num_cores=NC, num_subcores=NS
)


@functools.partial(
    pl.kernel,
    mesh=_mesh,
    out_type=jax.ShapeDtypeStruct((N_EDGES,), jnp.float32),
    compiler_params=pltpu.CompilerParams(needs_layout_passes=False),
    scratch_types=[
        pltpu.VMEM((N_NODES,), jnp.float32),
        pltpu.VMEM((2, WIN), jnp.int32),
        pltpu.VMEM((2, WIN), jnp.int32),
        pltpu.VMEM((CHUNK,), jnp.float32),
        pltpu.VMEM((CHUNK,), jnp.float32),
        pltpu.VMEM_SHARED((N_NODES,), jnp.float32),
        pltpu.SemaphoreType.DMA,
        pltpu.SemaphoreType.DMA,
        pltpu.SemaphoreType.DMA,
        pltpu.SemaphoreType.DMA,
    ],
)
def _edge_attn_sc(logits_hbm, ei_hbm, out_hbm,
                  tbl_v, idx_v0, idx_v1, out_v0, out_v1, attn_sh,
                  si0, si1, so0, so1):
    sid = lax.axis_index("s")
    wid = sid * NC + lax.axis_index("c")

    base_w = wid * E_PER_W
    idxs = (idx_v0, idx_v1)
    outs = (out_v0, out_v1)
    sin = (si0, si1)
    sout = (so0, so1)

    def start_in(b, c):
        base = base_w + c * CHUNK
        base_al = pl.multiple_of((base // 128) * 128, 128)
        pltpu.async_copy(ei_hbm.at[:, pl.ds(base_al, WIN)], idxs[b], sin[b])

    def wait_in(b):
        pltpu.make_async_copy(ei_hbm.at[:, pl.ds(0, WIN)], idxs[b], sin[b]).wait()

    def wait_out(b):
        pltpu.make_async_copy(outs[b], out_hbm.at[pl.ds(0, CHUNK)], sout[b]).wait()

    start_in(0, 0)

    s0 = pl.multiple_of(sid * SL, 8)
    pltpu.sync_copy(logits_hbm.at[pl.ds(s0, SL)], tbl_v.at[pl.ds(0, SL)])

    @plsc.parallel_loop(0, SL, step=L, unroll=4)
    def _sig(i):
        x = tbl_v[pl.ds(i, L)]
        tbl_v[pl.ds(i, L)] = 1.0 / (1.0 + jnp.exp(-x))

    pltpu.sync_copy(tbl_v.at[pl.ds(0, SL)], attn_sh.at[pl.ds(s0, SL)])

    @pl.when(sid == NS - 1)
    def _tail():
        pltpu.sync_copy(logits_hbm.at[pl.ds(SL * NS, TAIL)],
                        tbl_v.at[pl.ds(SL, TAIL)])

        @plsc.parallel_loop(SL, SL + TAIL, step=L, unroll=2)
        def _sig_t(i):
            x = tbl_v[pl.ds(i, L)]
            tbl_v[pl.ds(i, L)] = 1.0 / (1.0 + jnp.exp(-x))

        pltpu.sync_copy(tbl_v.at[pl.ds(SL, TAIL)],
                        attn_sh.at[pl.ds(SL * NS, TAIL)])

    plsc.subcore_barrier()
    pltpu.sync_copy(attn_sh, tbl_v)

    def pair_body(p, carry):
        for b in range(2):
            c = 2 * p + b
            cn = jnp.minimum(c + 1, N_CHUNKS - 1)
            start_in(1 - b, cn)
            wait_in(b)
            @pl.when(c >= 2)
            def _():
                wait_out(b)

            iv = idxs[b]
            ov = outs[b]
            base = base_w + c * CHUNK
            off = base - (base // 128) * 128

            @plsc.parallel_loop(0, CHUNK, step=L, unroll=10)
            def _gather(j):
                s = plsc.load_gather(tbl_v, [iv[0, pl.ds(off + j, L)]])
                d = plsc.load_gather(tbl_v, [iv[1, pl.ds(off + j, L)]])
                ov[pl.ds(j, L)] = s * d

            pltpu.async_copy(outs[b], out_hbm.at[pl.ds(base, CHUNK)], sout[b])
        return carry

    lax.fori_loop(0, N_CHUNKS // 2, pair_body, 0)
    wait_out(0)
    wait_out(1)
    wait_in(0)


def kernel(ver_logits, edge_index):
    return _edge_attn_sc(ver_logits, edge_index)
